# baseline (speedup 1.0000x reference)
import functools

import jax
import jax.numpy as jnp
from jax import lax
from jax.experimental import pallas as pl
from jax.experimental.pallas import tpu as pltpu
from jax.experimental.pallas import tpu_sc as plsc

NUM_EXPERTS = 8
TOP_K = 2
HIDDEN = 1024
INTER = 1024
TOKENS = 2048
TM = 256
NASSIGN = TOKENS * TOP_K
NBUF = NASSIGN + NUM_EXPERTS * TM
NTILES = NBUF // TM


def _routing_body(logits_ref, idx_ref, w_ref):
    logits = logits_ref[...]
    m = jnp.max(logits, axis=-1, keepdims=True)
    ex = jnp.exp(logits - m)
    probs = ex / jnp.sum(ex, axis=-1, keepdims=True)
    lanes = lax.broadcasted_iota(jnp.int32, probs.shape, 1)
    m1 = jnp.max(probs, axis=-1, keepdims=True)
    i1 = jnp.min(jnp.where(probs == m1, lanes, NUM_EXPERTS), axis=-1, keepdims=True)
    masked = jnp.where(lanes == i1, -jnp.inf, probs)
    m2 = jnp.max(masked, axis=-1, keepdims=True)
    i2 = jnp.min(jnp.where(masked == m2, lanes, NUM_EXPERTS), axis=-1, keepdims=True)
    denom = m1 + m2
    idx_ref[...] = jnp.where(lanes == 0, i1, jnp.where(lanes == 1, i2, 0))
    w_ref[...] = jnp.where(lanes == 0, m1 / denom, jnp.where(lanes == 1, m2 / denom, 0.0))


def _routing(router_logits):
    return pl.pallas_call(
        _routing_body,
        out_shape=(
            jax.ShapeDtypeStruct((TOKENS, NUM_EXPERTS), jnp.int32),
            jax.ShapeDtypeStruct((TOKENS, NUM_EXPERTS), jnp.float32),
        ),
    )(router_logits)


def _ffn_body(te_ref, xs_ref, w31_ref, w2_ref, wb_ref, out_ref):
    xs = xs_ref[...]
    proj = lax.dot_general(
        xs, w31_ref[0], (((1,), (1,)), ((), ())), preferred_element_type=jnp.float32
    )
    up = proj[:, :INTER]
    gate = proj[:, INTER:]
    h = gate * jax.nn.sigmoid(gate) * up
    y = lax.dot_general(
        h, w2_ref[0], (((1,), (1,)), ((), ())), preferred_element_type=jnp.float32
    )
    out_ref[...] = y * wb_ref[...]


def _ffn(xs, w3_w1_weight, w2_weight, wbuf, tile_eid):
    grid_spec = pltpu.PrefetchScalarGridSpec(
        num_scalar_prefetch=1,
        grid=(NTILES,),
        in_specs=[
            pl.BlockSpec((TM, HIDDEN), lambda i, te: (i, 0)),
            pl.BlockSpec((1, 2 * INTER, HIDDEN), lambda i, te: (te[i], 0, 0)),
            pl.BlockSpec((1, HIDDEN, INTER), lambda i, te: (te[i], 0, 0)),
            pl.BlockSpec((TM, 1), lambda i, te: (i, 0)),
        ],
        out_specs=pl.BlockSpec((TM, HIDDEN), lambda i, te: (i, 0)),
    )
    return pl.pallas_call(
        _ffn_body,
        grid_spec=grid_spec,
        out_shape=jax.ShapeDtypeStruct((NBUF, HIDDEN), jnp.float32),
        compiler_params=pltpu.CompilerParams(dimension_semantics=("arbitrary",)),
    )(tile_eid, xs, w3_w1_weight, w2_weight, wbuf.reshape(NBUF, 1))


def _dispatch(topk_idx, topk_w):
    e_flat = topk_idx[:, :TOP_K].reshape(-1)
    w_flat = topk_w[:, :TOP_K].reshape(-1)
    onehot = (e_flat[:, None] == jnp.arange(NUM_EXPERTS)[None, :]).astype(jnp.int32)
    csum = jnp.cumsum(onehot, axis=0)
    counts = csum[-1]
    rank = jnp.take_along_axis(csum, e_flat[:, None], axis=1)[:, 0] - 1
    padded = ((counts + TM - 1) // TM) * TM
    seg_end = jnp.cumsum(padded)
    offs = seg_end - padded
    slot = offs[e_flat] + rank
    src_tok = jnp.zeros((NBUF,), jnp.int32).at[slot].set(
        jnp.arange(NASSIGN, dtype=jnp.int32) // TOP_K)
    wbuf = jnp.zeros((NBUF,), jnp.float32).at[slot].set(w_flat)
    pos = slot.reshape(TOKENS, TOP_K)
    tile_starts = jnp.arange(NTILES, dtype=jnp.int32) * TM
    tile_eid = jnp.minimum(
        jnp.searchsorted(seg_end, tile_starts, side="right").astype(jnp.int32),
        NUM_EXPERTS - 1)
    return src_tok, wbuf, pos, tile_eid


def _sc_gather(x, src_tok):
    info = plsc.get_sparse_core_info()
    nw = info.num_cores * info.num_subcores
    b_per_w = NBUF // nw
    chunk = 64
    n_chunks = b_per_w // chunk
    mesh = plsc.VectorSubcoreMesh(core_axis_name="c", subcore_axis_name="s")

    @functools.partial(
        pl.kernel,
        mesh=mesh,
        out_type=jax.ShapeDtypeStruct((NBUF, HIDDEN), jnp.float32),
        scratch_types=[
            pltpu.VMEM((chunk,), jnp.int32),
            pltpu.VMEM((chunk, HIDDEN), jnp.float32),
            pltpu.SemaphoreType.DMA,
        ],
    )
    def k(x_hbm, idx_hbm, out_hbm, idx_v, rows_v, sem):
        wid = lax.axis_index("s") * info.num_cores + lax.axis_index("c")
        base = wid * b_per_w

        def body(c, _):
            off = base + c * chunk
            pltpu.sync_copy(idx_hbm.at[pl.ds(off, chunk)], idx_v)
            pltpu.async_copy(x_hbm.at[idx_v], rows_v, sem).wait()
            pltpu.sync_copy(rows_v, out_hbm.at[pl.ds(off, chunk)])
            return 0

        lax.fori_loop(0, n_chunks, body, 0)

    return k(x, src_tok)


def _sc_combine(ysw, pos0, pos1):
    info = plsc.get_sparse_core_info()
    nw = info.num_cores * info.num_subcores
    t_per_w = TOKENS // nw
    chunk = 32
    n_chunks = t_per_w // chunk
    mesh = plsc.VectorSubcoreMesh(core_axis_name="c", subcore_axis_name="s")

    @functools.partial(
        pl.kernel,
        mesh=mesh,
        out_type=jax.ShapeDtypeStruct((TOKENS, HIDDEN), jnp.float32),
        scratch_types=[
            pltpu.VMEM((chunk,), jnp.int32),
            pltpu.VMEM((chunk,), jnp.int32),
            pltpu.VMEM((chunk, HIDDEN), jnp.float32),
            pltpu.VMEM((chunk, HIDDEN), jnp.float32),
            pltpu.SemaphoreType.DMA,
            pltpu.SemaphoreType.DMA,
        ],
    )
    def k(ysw_hbm, p0_hbm, p1_hbm, out_hbm, i0_v, i1_v, r0_v, r1_v, s0, s1):
        wid = lax.axis_index("s") * info.num_cores + lax.axis_index("c")
        base = wid * t_per_w

        def body(c, _):
            off = base + c * chunk
            pltpu.sync_copy(p0_hbm.at[pl.ds(off, chunk)], i0_v)
            pltpu.sync_copy(p1_hbm.at[pl.ds(off, chunk)], i1_v)
            cp0 = pltpu.async_copy(ysw_hbm.at[i0_v], r0_v, s0)
            cp1 = pltpu.async_copy(ysw_hbm.at[i1_v], r1_v, s1)
            cp0.wait()
            cp1.wait()

            def row(i, _):
                for j in range(HIDDEN // 16):
                    sl = pl.ds(j * 16, 16)
                    r0_v[i, sl] = r0_v[i, sl] + r1_v[i, sl]
                return 0

            lax.fori_loop(0, chunk, row, 0)
            pltpu.sync_copy(r0_v, out_hbm.at[pl.ds(off, chunk)])
            return 0

        lax.fori_loop(0, n_chunks, body, 0)

    return k(ysw, pos0, pos1)


def kernel(x, router_logits, w3_w1_weight, w2_weight):
    topk_idx, topk_w = _routing(router_logits)
    src_tok, wbuf, pos, tile_eid = _dispatch(topk_idx, topk_w)
    xs = _sc_gather(x, src_tok)
    ysw = _ffn(xs, w3_w1_weight, w2_weight, wbuf, tile_eid)
    out = _sc_combine(ysw, pos[:, 0].copy(), pos[:, 1].copy())
    return out.astype(x.dtype)
